# Initial kernel scaffold; baseline (speedup 1.0000x reference)
import jax
import jax.numpy as jnp
from jax.experimental import pallas as pl


def kernel(roads, dict_u, dict_s1, dict_s2, dict_s3, Wu, Ws1, Ws2, Ws3, W1, b1, W21, b21, W22, b22):
    raise NotImplementedError("write your pallas kernel here")


# speedup vs baseline: 33.2378x; 33.2378x over previous
import functools

import jax
import jax.numpy as jnp
from jax import lax
from jax.experimental import pallas as pl
from jax.experimental.pallas import tpu as pltpu
from jax.experimental.pallas import tpu_sc as plsc

_SC_PARAMS = pltpu.CompilerParams(use_tc_tiling_on_sc=False)

_NC = 2
_NS = 16
_NW = _NC * _NS
_CHUNK = 128


def _sc_mesh():
    return plsc.VectorSubcoreMesh(core_axis_name="c", subcore_axis_name="s")


def _worker_id():
    return lax.axis_index("s") * _NC + lax.axis_index("c")


def _build_u_gather(n_ids, d_u):
    @functools.partial(
        pl.kernel,
        mesh=_sc_mesh(),
        out_type=jax.ShapeDtypeStruct((n_ids, d_u), jnp.float32),
        scratch_types=[
            pltpu.VMEM((n_ids,), jnp.int32),
            pltpu.VMEM((n_ids, d_u), jnp.float32),
            pltpu.SemaphoreType.DMA,
        ],
        compiler_params=_SC_PARAMS,
    )
    def u_gather(dict_u_hbm, wu_hbm, out_hbm, idx_v, rows_v, sem):
        @pl.when(_worker_id() == 0)
        def _():
            pltpu.sync_copy(dict_u_hbm, idx_v)
            pltpu.async_copy(wu_hbm.at[idx_v], rows_v, sem).wait()
            pltpu.sync_copy(rows_v, out_hbm)

    return u_gather


def _build_table_mlp(n_ids, d_u, n1, d1, n2, d2, n3, d3, h_dim, d_out):
    def body(u_tab, s1_ids, s2_ids, s3_ids, ws1, ws2, ws3,
             w1ut, w1s1t, w1s2t, w1s3t, b1, w21t, b21, out):
        f32 = jnp.float32
        dot = functools.partial(jnp.dot, preferred_element_type=f32,
                                precision=lax.Precision.HIGHEST)

        def onehot(ids_ref, n):
            ids = ids_ref[...]
            cols = lax.broadcasted_iota(jnp.int32, (n_ids, n), 1)
            return (ids == cols).astype(f32)

        s1 = dot(onehot(s1_ids, n1), ws1[...])
        s2 = dot(onehot(s2_ids, n2), ws2[...])
        s3 = dot(onehot(s3_ids, n3), ws3[...])
        h = (dot(u_tab[...], w1ut[...])
             + dot(s1, w1s1t[...])
             + dot(s2, w1s2t[...])
             + dot(s3, w1s3t[...])
             + b1[...])
        h = jnp.maximum(h, 0.0)
        out[...] = dot(h, w21t[...]) + b21[...]

    return pl.pallas_call(
        body,
        out_shape=jax.ShapeDtypeStruct((n_ids, d_out), jnp.float32),
    )


def _build_table_gather(n_rows, d_out, n_ids):
    chunks_per_w = n_rows // (_NW * _CHUNK)

    @functools.partial(
        pl.kernel,
        mesh=_sc_mesh(),
        out_type=jax.ShapeDtypeStruct((n_rows, d_out), jnp.float32),
        scratch_types=[
            pltpu.VMEM((chunks_per_w, _CHUNK), jnp.int32),
            pltpu.VMEM((_CHUNK, d_out), jnp.float32),
            pltpu.VMEM((_CHUNK, d_out), jnp.float32),
            pltpu.SemaphoreType.DMA,
            pltpu.SemaphoreType.DMA,
        ],
        compiler_params=_SC_PARAMS,
    )
    def table_gather(roads_hbm, table_hbm, out_hbm, idx_v, buf0, buf1,
                     sem0, sem1):
        wid = _worker_id()
        chunk0 = wid * chunks_per_w
        pltpu.sync_copy(roads_hbm.at[pl.ds(chunk0, chunks_per_w)], idx_v)

        pltpu.make_async_copy(table_hbm.at[idx_v.at[0]], buf0, sem0).start()

        def pair_body(p, carry):
            j0 = p * 2
            pltpu.make_async_copy(table_hbm.at[idx_v.at[j0 + 1]], buf1, sem1).start()
            pltpu.make_async_copy(table_hbm.at[idx_v.at[j0]], buf0, sem0).wait()
            pltpu.sync_copy(buf0, out_hbm.at[pl.ds((chunk0 + j0) * _CHUNK, _CHUNK)])
            @pl.when(p + 1 < chunks_per_w // 2)
            def _():
                pltpu.make_async_copy(table_hbm.at[idx_v.at[j0 + 2]], buf0, sem0).start()
            pltpu.make_async_copy(table_hbm.at[idx_v.at[j0 + 1]], buf1, sem1).wait()
            pltpu.sync_copy(buf1, out_hbm.at[pl.ds((chunk0 + j0 + 1) * _CHUNK, _CHUNK)])
            return carry

        lax.fori_loop(0, chunks_per_w // 2, pair_body, 0)

    return table_gather


def kernel(roads, dict_u, dict_s1, dict_s2, dict_s3, Wu, Ws1, Ws2, Ws3,
           W1, b1, W21, b21, W22, b22):
    del W22, b22
    f32 = jnp.float32
    roads = roads.astype(jnp.int32)
    B, S = roads.shape
    n_ids = dict_u.shape[0]
    d_u = Wu.shape[1]
    n1, d1 = Ws1.shape
    n2, d2 = Ws2.shape
    n3, d3 = Ws3.shape
    h_dim = W1.shape[0]
    d_out = W21.shape[0]
    n_rows = B * S

    u_tab = _build_u_gather(n_ids, d_u)(
        dict_u.astype(jnp.int32), Wu.astype(f32))

    w1t = W1.astype(f32).T
    table = _build_table_mlp(n_ids, d_u, n1, d1, n2, d2, n3, d3, h_dim, d_out)(
        u_tab,
        dict_s1.astype(jnp.int32).reshape(n_ids, 1),
        dict_s2.astype(jnp.int32).reshape(n_ids, 1),
        dict_s3.astype(jnp.int32).reshape(n_ids, 1),
        Ws1.astype(f32), Ws2.astype(f32), Ws3.astype(f32),
        w1t[:d_u], w1t[d_u:d_u + d1], w1t[d_u + d1:d_u + d1 + d2],
        w1t[d_u + d1 + d2:],
        b1.astype(f32).reshape(1, h_dim),
        W21.astype(f32).T,
        b21.astype(f32).reshape(1, d_out),
    )

    roads2d = roads.reshape(n_rows // _CHUNK, _CHUNK)
    out = _build_table_gather(n_rows, d_out, n_ids)(roads2d, table)
    return out.reshape(B, S, d_out)
